# baseline (speedup 1.0000x reference)
import functools

import jax
import jax.numpy as jnp
from jax import lax
from jax.experimental import pallas as pl
from jax.experimental.pallas import tpu as pltpu
from jax.experimental.pallas import tpu_sc as plsc

N = 2048
E = 65536
D_IN = 256
R = 3
HID = 128
LAT = 64
HEADS = 4

ROW_BLK = 256


def _adj_pred_body(z_blk_ref, zt_ref, rd_ref, out_ref):
    z_blk = z_blk_ref[...]
    zt = zt_ref[...]
    for r in range(R):
        zr = z_blk * rd_ref[r][None, :]
        score = jnp.dot(zr, zt, preferred_element_type=jnp.float32)
        score = jnp.clip(score, -10.0, 10.0)
        p = jax.nn.sigmoid(score)
        out_ref[r, 0] = jnp.clip(p, 1e-6, 1.0 - 1e-6)


def _adj_preds(z, rel_diag):
    zt = z.T
    grid = (N // ROW_BLK,)
    return pl.pallas_call(
        _adj_pred_body,
        grid=grid,
        in_specs=[
            pl.BlockSpec((ROW_BLK, LAT), lambda i: (i, 0)),
            pl.BlockSpec((LAT, N), lambda i: (0, 0)),
            pl.BlockSpec((R, LAT), lambda i: (0, 0)),
        ],
        out_specs=pl.BlockSpec((R, 1, ROW_BLK, N), lambda i: (0, 0, i, 0)),
        out_shape=jax.ShapeDtypeStruct((R, 1, N, N), jnp.float32),
    )(z, zt, rel_diag)


def _edge_addr_body(src_ref, dst_ref, et_ref, f_ref, sidx_ref, didx_ref):
    base = et_ref[...] * N
    sidx_ref[...] = base + src_ref[...]
    didx_ref[...] = base + dst_ref[...]
    f_ref[...] = sidx_ref[...] * N + dst_ref[...]


def _edge_addrs(src, dst, et):
    s2 = src.reshape(64, 1024)
    d2 = dst.reshape(64, 1024)
    t2 = et.reshape(64, 1024)
    f, sidx, didx = pl.pallas_call(
        _edge_addr_body,
        out_shape=[jax.ShapeDtypeStruct((64, 1024), jnp.int32)] * 3,
    )(s2, d2, t2)
    return f.reshape(E), sidx.reshape(512, 128), didx.reshape(512, 128)


_ADJ_PAGE = 65536
_ADJ_ROUNDS = (R * N * N) // (32 * _ADJ_PAGE)
_EBLK = 8192


def _adj_true_body(f_hbm, out_hbm, page, fbufs, sem0, sem1):
    wid = lax.axis_index("s") * 2 + lax.axis_index("c")
    ones = jnp.ones((16,), jnp.float32)
    zeros16 = jnp.zeros((16,), jnp.float32)

    def round_body(k, _):
        base = (k * 32 + wid) * _ADJ_PAGE

        def zero_chunk(i, _):
            for u in range(8):
                page[(i * 128 + u * 16) // 2048, pl.ds((i * 128 + u * 16) % 2048, 16)] = zeros16
            return _
        lax.fori_loop(0, _ADJ_PAGE // 128, zero_chunk, None)
        cp0 = pltpu.async_copy(f_hbm.at[pl.ds(0, _EBLK)], fbufs.at[0], sem0)
        for b in range(E // _EBLK):
            cur = b % 2
            if b % 2 == 0:
                cp0.wait()
            else:
                cp1.wait()
            if b + 1 < E // _EBLK:
                nxt = (b + 1) % 2
                if nxt == 0:
                    cp0 = pltpu.async_copy(
                        f_hbm.at[pl.ds((b + 1) * _EBLK, _EBLK)],
                        fbufs.at[0], sem0)
                else:
                    cp1 = pltpu.async_copy(
                        f_hbm.at[pl.ds((b + 1) * _EBLK, _EBLK)],
                        fbufs.at[1], sem1)

            def chunk(i, _):
                for u in range(8):
                    fv = fbufs[cur, pl.ds(i * 128 + u * 16, 16)]
                    local = fv - base
                    m = plsc.bitcast(local, jnp.uint32) < jnp.uint32(_ADJ_PAGE)
                    plsc.addupdate_scatter(
                        page, [local >> 11, local & 2047], ones, mask=m)
                return _
            lax.fori_loop(0, _EBLK // 128, chunk, None)
        rbase = k * 1024 + wid * 32
        rel = rbase // N
        srow = rbase - rel * N
        pltpu.sync_copy(page, out_hbm.at[rel, 0, pl.ds(srow, 32), :])
        return _
    lax.fori_loop(0, _ADJ_ROUNDS, round_body, None)


def _adj_true(f):
    mesh = plsc.VectorSubcoreMesh(core_axis_name="c", subcore_axis_name="s")
    run = pl.kernel(
        _adj_true_body,
        out_type=jax.ShapeDtypeStruct((R, 1, N, N), jnp.float32),
        mesh=mesh,
        compiler_params=pltpu.CompilerParams(needs_layout_passes=False),
        scratch_types=[
            pltpu.VMEM((32, N), jnp.float32),
            pltpu.VMEM((2, _EBLK), jnp.int32),
            pltpu.SemaphoreType.DMA,
            pltpu.SemaphoreType.DMA,
        ],
    )
    return run(f)


_EPT = E // 32
_NCH = _EPT // 128


def _edge_conv_body(C, H, sidx_hbm, didx_hbm, dst_hbm, qd_hbm, ks_hbm,
                    xr_hbm, num_out, den_out, qd_v, ks_v, den_v, sidx_v,
                    didx_v, dst_v, wbuf, rows, num_sh, sem):
    cid = lax.axis_index("c")
    sid = lax.axis_index("s")
    wid = sid * 2 + cid
    zeros16 = jnp.zeros((16,), jnp.float32)
    JPH = 32 // 16 if H > 1 else C // 16

    pltpu.sync_copy(qd_hbm, qd_v)
    pltpu.sync_copy(ks_hbm, ks_v)
    pltpu.sync_copy(sidx_hbm.at[pl.ds(wid * 16, 16)], sidx_v)
    pltpu.sync_copy(didx_hbm.at[pl.ds(wid * 16, 16)], didx_v)
    pltpu.sync_copy(dst_hbm.at[pl.ds(wid * 16, 16)], dst_v)

    def zden(i, _):
        den_v[pl.ds(i * 16, 16)] = zeros16
        return _
    lax.fori_loop(0, N * H // 16, zden, None)

    def zrow(r_, _):
        for u in range(C // 16):
            rows[r_, pl.ds(u * 16, 16)] = zeros16
        return _
    lax.fori_loop(0, 128, zrow, None)
    pltpu.sync_copy(rows, num_sh.at[pl.ds(sid * 128, 128)])
    plsc.subcore_barrier()

    def chunk(mc, _):
        pltpu.async_copy(xr_hbm.at[sidx_v.at[mc]], rows, sem).wait()
        for g in range(8):
            iv = sidx_v[mc, pl.ds(g * 16, 16)]
            jv = didx_v[mc, pl.ds(g * 16, 16)]
            dn = dst_v[mc, pl.ds(g * 16, 16)]
            for h in range(H):
                hf = jnp.full((16,), h, jnp.int32)
                q = plsc.load_gather(qd_v, [jv * H + hf])
                k = plsc.load_gather(ks_v, [iv * H + hf])
                l = q + k
                l = jnp.where(l >= 0.0, l, 0.2 * l)
                w = jnp.exp(l)
                wbuf[h, pl.ds(g * 16, 16)] = w
                plsc.addupdate_scatter(den_v, [dn * H + hf], w)
        def scale(e, _):
            ef = jnp.full((16,), e, jnp.int32)
            for h in range(H):
                hf = jnp.full((16,), h, jnp.int32)
                bc = plsc.load_gather(wbuf, [hf, ef])
                for u in range(JPH):
                    sl = pl.ds(h * (C // H) + u * 16, 16)
                    rows[e, sl] = rows[e, sl] * bc
            return _
        lax.fori_loop(0, 128, scale, None)
        pltpu.sync_copy(rows, num_sh.at[dst_v.at[mc]], add=True)
        return _
    lax.fori_loop(0, _NCH, chunk, None)

    plsc.subcore_barrier()
    pltpu.sync_copy(num_sh.at[pl.ds(sid * 128, 128)],
                    num_out.at[cid, pl.ds(sid * 128, 128)])
    pltpu.sync_copy(den_v, den_out.at[pl.ds(wid * N * H, N * H)])


def _edge_conv(C, H):
    mesh = plsc.VectorSubcoreMesh(core_axis_name="c", subcore_axis_name="s")
    body = functools.partial(_edge_conv_body, C, H)
    return pl.kernel(
        body,
        out_type=[
            jax.ShapeDtypeStruct((2, N, C), jnp.float32),
            jax.ShapeDtypeStruct((32 * N * H,), jnp.float32),
        ],
        mesh=mesh,
        compiler_params=pltpu.CompilerParams(needs_layout_passes=False),
        scratch_types=[
            pltpu.VMEM((R * N * H,), jnp.float32),
            pltpu.VMEM((R * N * H,), jnp.float32),
            pltpu.VMEM((N * H,), jnp.float32),
            pltpu.VMEM((16, 128), jnp.int32),
            pltpu.VMEM((16, 128), jnp.int32),
            pltpu.VMEM((16, 128), jnp.int32),
            pltpu.VMEM((H, 128), jnp.float32),
            pltpu.VMEM((128, C), jnp.float32),
            pltpu.VMEM_SHARED((N, C), jnp.float32),
            pltpu.SemaphoreType.DMA,
        ],
    )


def _rgat_sc(xrflat, q, k, sidx2, didx2, dst2, C, H):
    xr4 = xrflat.reshape(R * N, H, C // H)
    qd = jnp.einsum('nhc,hc->nh', xr4, q.reshape(H, C // H)).reshape(R * N * H)
    ks = jnp.einsum('nhc,hc->nh', xr4, k.reshape(H, C // H)).reshape(R * N * H)
    Cp = C
    if H == 1 and C < 128:
        Cp = 128
        xrflat = jnp.pad(xrflat, ((0, 0), (0, Cp - C)))
    num_parts, den_parts = _edge_conv(Cp, H)(
        sidx2, didx2, dst2, qd, ks, xrflat)
    num = num_parts[0] + num_parts[1]
    den = den_parts.reshape(32, N, H).sum(0)
    out = num.reshape(N, H, Cp // H) / (den[:, :, None] + 1e-16)
    return out.reshape(N, Cp)[:, :C]


_NB = 256
_NG = N // _NB


def _enc1_body(x_ref, W1_ref, aq_ref, ak_ref, xr_ref, qd_ref, ks_ref):
    xb = x_ref[...]
    for r in range(R):
        xr = jnp.dot(xb, W1_ref[r], preferred_element_type=jnp.float32)
        xr_ref[r] = xr
        qd_ref[r] = jnp.dot(xr, aq_ref[...], preferred_element_type=jnp.float32)
        ks_ref[r] = jnp.dot(xr, ak_ref[...], preferred_element_type=jnp.float32)


def _enc1(x, W1, Aq, Ak):
    return pl.pallas_call(
        _enc1_body,
        grid=(_NG,),
        in_specs=[
            pl.BlockSpec((_NB, D_IN), lambda i: (i, 0)),
            pl.BlockSpec((R, D_IN, HID), lambda i: (0, 0, 0)),
            pl.BlockSpec((HID, HEADS), lambda i: (0, 0)),
            pl.BlockSpec((HID, HEADS), lambda i: (0, 0)),
        ],
        out_specs=[
            pl.BlockSpec((R, _NB, HID), lambda i: (0, i, 0)),
            pl.BlockSpec((R, _NB, HEADS), lambda i: (0, i, 0)),
            pl.BlockSpec((R, _NB, HEADS), lambda i: (0, i, 0)),
        ],
        out_shape=[
            jax.ShapeDtypeStruct((R, N, HID), jnp.float32),
            jax.ShapeDtypeStruct((R, N, HEADS), jnp.float32),
            jax.ShapeDtypeStruct((R, N, HEADS), jnp.float32),
        ],
    )(x, W1, Aq, Ak)


def _bnstat_body(nump_ref, denp_ref, exp_ref, out1_ref, psum_ref, psq_ref):
    i = pl.program_id(0)
    num = nump_ref[0] + nump_ref[1]
    den = jnp.sum(denp_ref[...], axis=0)
    den128 = jnp.dot(den, exp_ref[...],
                     preferred_element_type=jnp.float32) + 1e-16
    out1 = num / den128
    out1_ref[...] = out1

    @pl.when(i == 0)
    def _():
        psum_ref[...] = jnp.zeros_like(psum_ref)
        psq_ref[...] = jnp.zeros_like(psq_ref)
    psum_ref[...] += jnp.sum(out1, axis=0, keepdims=True)
    psq_ref[...] += jnp.sum(out1 * out1, axis=0, keepdims=True)


def _bnstat(num_parts, den_parts, expand):
    return pl.pallas_call(
        _bnstat_body,
        grid=(_NG,),
        in_specs=[
            pl.BlockSpec((2, _NB, HID), lambda i: (0, i, 0)),
            pl.BlockSpec((32, _NB, HEADS), lambda i: (0, i, 0)),
            pl.BlockSpec((HEADS, HID), lambda i: (0, 0)),
        ],
        out_specs=[
            pl.BlockSpec((_NB, HID), lambda i: (i, 0)),
            pl.BlockSpec((1, HID), lambda i: (0, 0)),
            pl.BlockSpec((1, HID), lambda i: (0, 0)),
        ],
        out_shape=[
            jax.ShapeDtypeStruct((N, HID), jnp.float32),
            jax.ShapeDtypeStruct((1, HID), jnp.float32),
            jax.ShapeDtypeStruct((1, HID), jnp.float32),
        ],
    )(num_parts, den_parts, expand)


def _enc2_body(out1_ref, psum_ref, psq_ref, g_ref, b_ref, W2_ref, aq_ref,
               ak_ref, rw_ref, h_ref, xr2_ref, qd2_ref, ks2_ref, hres_ref):
    mu = psum_ref[...] / N
    var = psq_ref[...] / N - mu * mu
    hn = (out1_ref[...] - mu) / jnp.sqrt(var + 1e-5) * g_ref[...] + b_ref[...]
    h = jnp.where(hn >= 0.0, hn, 0.2 * hn)
    h_ref[...] = h
    for r in range(R):
        xr = jnp.dot(h, W2_ref[r], preferred_element_type=jnp.float32)
        xr2_ref[r] = xr
        qd2_ref[r] = jnp.dot(xr, aq_ref[...], preferred_element_type=jnp.float32)
        ks2_ref[r] = jnp.dot(xr, ak_ref[...], preferred_element_type=jnp.float32)
    hres_ref[...] = jnp.dot(h, rw_ref[...], preferred_element_type=jnp.float32)


def _enc2(out1, psum, psq, g1, b1, W2p, Aq2, Ak2, res_W):
    return pl.pallas_call(
        _enc2_body,
        grid=(_NG,),
        in_specs=[
            pl.BlockSpec((_NB, HID), lambda i: (i, 0)),
            pl.BlockSpec((1, HID), lambda i: (0, 0)),
            pl.BlockSpec((1, HID), lambda i: (0, 0)),
            pl.BlockSpec((1, HID), lambda i: (0, 0)),
            pl.BlockSpec((1, HID), lambda i: (0, 0)),
            pl.BlockSpec((R, HID, 128), lambda i: (0, 0, 0)),
            pl.BlockSpec((128, 8), lambda i: (0, 0)),
            pl.BlockSpec((128, 8), lambda i: (0, 0)),
            pl.BlockSpec((HID, LAT), lambda i: (0, 0)),
        ],
        out_specs=[
            pl.BlockSpec((_NB, HID), lambda i: (i, 0)),
            pl.BlockSpec((R, _NB, 128), lambda i: (0, i, 0)),
            pl.BlockSpec((R, _NB, 8), lambda i: (0, i, 0)),
            pl.BlockSpec((R, _NB, 8), lambda i: (0, i, 0)),
            pl.BlockSpec((_NB, LAT), lambda i: (i, 0)),
        ],
        out_shape=[
            jax.ShapeDtypeStruct((N, HID), jnp.float32),
            jax.ShapeDtypeStruct((R, N, 128), jnp.float32),
            jax.ShapeDtypeStruct((R, N, 8), jnp.float32),
            jax.ShapeDtypeStruct((R, N, 8), jnp.float32),
            jax.ShapeDtypeStruct((N, LAT), jnp.float32),
        ],
    )(out1, psum, psq, g1.reshape(1, HID), b1.reshape(1, HID), W2p, Aq2,
      Ak2, res_W)


def _dec_body(nump_ref, denp_ref, hres_ref, rb_ref, dW1_ref, db1_ref,
              lng_ref, lnb_ref, dW2_ref, db2_ref, z_ref, xhat_ref, gmax_ref):
    i = pl.program_id(0)
    num = (nump_ref[0] + nump_ref[1])[:, :LAT]
    den = jnp.sum(denp_ref[...], axis=0)[:, None] + 1e-16
    zc = jnp.clip(num / den, -10.0, 10.0)
    z = zc + hres_ref[...] + rb_ref[...]
    z_ref[...] = z
    t = jnp.dot(z, dW1_ref[...], preferred_element_type=jnp.float32) + db1_ref[...]
    tm = jnp.mean(t, axis=1, keepdims=True)
    tv = jnp.mean(t * t, axis=1, keepdims=True) - tm * tm
    t = (t - tm) / jnp.sqrt(tv + 1e-5) * lng_ref[...] + lnb_ref[...]
    t = jnp.maximum(t, 0.0)
    xhat_ref[...] = jnp.dot(t, dW2_ref[...],
                            preferred_element_type=jnp.float32) + db2_ref[...]
    bmax = jnp.max(z, axis=0, keepdims=True)

    @pl.when(i == 0)
    def _():
        gmax_ref[...] = bmax

    @pl.when(i > 0)
    def _():
        gmax_ref[...] = jnp.maximum(gmax_ref[...], bmax)


def _dec(num_parts, den_parts, hres, res_b, dec_W1, dec_b1, ln_g, ln_b,
         dec_W2, dec_b2):
    return pl.pallas_call(
        _dec_body,
        grid=(_NG,),
        in_specs=[
            pl.BlockSpec((2, _NB, 128), lambda i: (0, i, 0)),
            pl.BlockSpec((32, _NB), lambda i: (0, i)),
            pl.BlockSpec((_NB, LAT), lambda i: (i, 0)),
            pl.BlockSpec((1, LAT), lambda i: (0, 0)),
            pl.BlockSpec((LAT, HID), lambda i: (0, 0)),
            pl.BlockSpec((1, HID), lambda i: (0, 0)),
            pl.BlockSpec((1, HID), lambda i: (0, 0)),
            pl.BlockSpec((1, HID), lambda i: (0, 0)),
            pl.BlockSpec((HID, D_IN), lambda i: (0, 0)),
            pl.BlockSpec((1, D_IN), lambda i: (0, 0)),
        ],
        out_specs=[
            pl.BlockSpec((_NB, LAT), lambda i: (i, 0)),
            pl.BlockSpec((_NB, D_IN), lambda i: (i, 0)),
            pl.BlockSpec((1, LAT), lambda i: (0, 0)),
        ],
        out_shape=[
            jax.ShapeDtypeStruct((N, LAT), jnp.float32),
            jax.ShapeDtypeStruct((N, D_IN), jnp.float32),
            jax.ShapeDtypeStruct((1, LAT), jnp.float32),
        ],
    )(num_parts, den_parts, hres, res_b.reshape(1, LAT), dec_W1,
      dec_b1.reshape(1, HID), ln_g.reshape(1, HID), ln_b.reshape(1, HID),
      dec_W2, dec_b2.reshape(1, D_IN))


def kernel(x, edge_index, edge_type, W1, q1, k1, g1, b1, W2, q2, k2,
           res_W, res_b, dec_W1, dec_b1, ln_g, ln_b, dec_W2, dec_b2,
           rel_diag):
    src = edge_index[0]
    dst = edge_index[1]
    f, sidx2, didx2 = _edge_addrs(src, dst, edge_type)
    dst2 = dst.reshape(512, 128)

    eye4 = jnp.eye(HEADS, dtype=jnp.float32)
    Aq1 = (q1[:, :, None] * eye4[:, None, :]).reshape(HID, HEADS)
    Ak1 = (k1[:, :, None] * eye4[:, None, :]).reshape(HID, HEADS)
    xr1, qd1, ks1 = _enc1(x, W1, Aq1, Ak1)
    num1p, den1p = _edge_conv(HID, HEADS)(
        sidx2, didx2, dst2, qd1.reshape(R * N * HEADS),
        ks1.reshape(R * N * HEADS), xr1.reshape(R * N, HID))

    expand = jnp.repeat(eye4, HID // HEADS, axis=1)
    out1, psum, psq = _bnstat(num1p, den1p.reshape(32, N, HEADS), expand)

    W2p = jnp.pad(W2, ((0, 0), (0, 0), (0, 128 - LAT)))
    Aq2 = jnp.zeros((128, 8), jnp.float32).at[:LAT, 0].set(q2[0])
    Ak2 = jnp.zeros((128, 8), jnp.float32).at[:LAT, 0].set(k2[0])
    h, xr2p, qd2, ks2, hres = _enc2(out1, psum, psq, g1, b1, W2p, Aq2,
                                    Ak2, res_W)
    num2p, den2p = _edge_conv(128, 1)(
        sidx2, didx2, dst2, qd2[:, :, 0].reshape(R * N),
        ks2[:, :, 0].reshape(R * N), xr2p.reshape(R * N, 128))

    z, x_hat, graph_embedding = _dec(
        num2p, den2p.reshape(32, N), hres, res_b, dec_W1, dec_b1, ln_g,
        ln_b, dec_W2, dec_b2)

    mask = jnp.ones((1, N), dtype=bool)
    adj_preds = _adj_preds(z, rel_diag)
    adj_true_rel = _adj_true(f)
    return (z, x_hat, adj_preds, adj_true_rel, mask, graph_embedding)


# speedup vs baseline: 1.2657x; 1.2657x over previous
import functools

import jax
import jax.numpy as jnp
from jax import lax
from jax.experimental import pallas as pl
from jax.experimental.pallas import tpu as pltpu
from jax.experimental.pallas import tpu_sc as plsc

N = 2048
E = 65536
D_IN = 256
R = 3
HID = 128
LAT = 64
HEADS = 4

ROW_BLK = 256


def _adj_pred_body(z_blk_ref, zt_ref, rd_ref, out_ref):
    z_blk = z_blk_ref[...]
    zt = zt_ref[...]
    for r in range(R):
        zr = z_blk * rd_ref[r][None, :]
        score = jnp.dot(zr, zt, preferred_element_type=jnp.float32)
        score = jnp.clip(score, -10.0, 10.0)
        p = jax.nn.sigmoid(score)
        out_ref[r, 0] = jnp.clip(p, 1e-6, 1.0 - 1e-6)


def _adj_preds(z, rel_diag):
    zt = z.T
    grid = (N // ROW_BLK,)
    return pl.pallas_call(
        _adj_pred_body,
        grid=grid,
        in_specs=[
            pl.BlockSpec((ROW_BLK, LAT), lambda i: (i, 0)),
            pl.BlockSpec((LAT, N), lambda i: (0, 0)),
            pl.BlockSpec((R, LAT), lambda i: (0, 0)),
        ],
        out_specs=pl.BlockSpec((R, 1, ROW_BLK, N), lambda i: (0, 0, i, 0)),
        out_shape=jax.ShapeDtypeStruct((R, 1, N, N), jnp.float32),
    )(z, zt, rel_diag)


def _edge_addr_body(src_ref, dst_ref, et_ref, f_ref, sidx_ref, didx_ref):
    base = et_ref[...] * N
    sidx_ref[...] = base + src_ref[...]
    didx_ref[...] = base + dst_ref[...]
    f_ref[...] = sidx_ref[...] * N + dst_ref[...]


def _edge_addrs(src, dst, et):
    s2 = src.reshape(64, 1024)
    d2 = dst.reshape(64, 1024)
    t2 = et.reshape(64, 1024)
    f, sidx, didx = pl.pallas_call(
        _edge_addr_body,
        out_shape=[jax.ShapeDtypeStruct((64, 1024), jnp.int32)] * 3,
    )(s2, d2, t2)
    return f.reshape(E), sidx.reshape(512, 128), didx.reshape(512, 128)


_ADJ_ROWS = 48
_ADJ_PAGE = _ADJ_ROWS * N
_ADJ_ROUNDS = (R * N * N) // (32 * _ADJ_PAGE)
_EBLK = 8192


def _adj_true_body(f_hbm, out_hbm, page, fbufs, sem0, sem1):
    wid = lax.axis_index("s") * 2 + lax.axis_index("c")
    ones = jnp.ones((16,), jnp.float32)
    zeros16 = jnp.zeros((16,), jnp.float32)

    def round_body(k, _):
        base = (k * 32 + wid) * _ADJ_PAGE

        def zero_chunk(i, _):
            for u in range(8):
                page[(i * 128 + u * 16) // 2048, pl.ds((i * 128 + u * 16) % 2048, 16)] = zeros16
            return _
        lax.fori_loop(0, _ADJ_PAGE // 128, zero_chunk, None)
        cp0 = pltpu.async_copy(f_hbm.at[pl.ds(0, _EBLK)], fbufs.at[0], sem0)
        for b in range(E // _EBLK):
            cur = b % 2
            if b % 2 == 0:
                cp0.wait()
            else:
                cp1.wait()
            if b + 1 < E // _EBLK:
                nxt = (b + 1) % 2
                if nxt == 0:
                    cp0 = pltpu.async_copy(
                        f_hbm.at[pl.ds((b + 1) * _EBLK, _EBLK)],
                        fbufs.at[0], sem0)
                else:
                    cp1 = pltpu.async_copy(
                        f_hbm.at[pl.ds((b + 1) * _EBLK, _EBLK)],
                        fbufs.at[1], sem1)

            def chunk(i, _):
                for u in range(8):
                    fv = fbufs[cur, pl.ds(i * 128 + u * 16, 16)]
                    local = fv - base
                    m = plsc.bitcast(local, jnp.uint32) < jnp.uint32(_ADJ_PAGE)
                    plsc.addupdate_scatter(
                        page, [local >> 11, local & 2047], ones, mask=m)
                return _
            lax.fori_loop(0, _EBLK // 128, chunk, None)
        rbase = (k * 32 + wid) * _ADJ_ROWS
        for sb in range(_ADJ_ROWS // 16):
            rb = rbase + sb * 16
            rel = rb // N
            srow = rb - rel * N
            pltpu.sync_copy(page.at[pl.ds(sb * 16, 16)],
                            out_hbm.at[rel, 0, pl.ds(srow, 16), :])
        return _
    lax.fori_loop(0, _ADJ_ROUNDS, round_body, None)


def _adj_true(f):
    mesh = plsc.VectorSubcoreMesh(core_axis_name="c", subcore_axis_name="s")
    run = pl.kernel(
        _adj_true_body,
        out_type=jax.ShapeDtypeStruct((R, 1, N, N), jnp.float32),
        mesh=mesh,
        compiler_params=pltpu.CompilerParams(needs_layout_passes=False),
        scratch_types=[
            pltpu.VMEM((_ADJ_ROWS, N), jnp.float32),
            pltpu.VMEM((2, _EBLK), jnp.int32),
            pltpu.SemaphoreType.DMA,
            pltpu.SemaphoreType.DMA,
        ],
    )
    return run(f)


_EPT = E // 32
_NCH = _EPT // 128


def _edge_conv_body(C, H, sidx_hbm, didx_hbm, dst_hbm, qd_hbm, ks_hbm,
                    xr_hbm, num_out, den_out, qd_v, ks_v, den_v, sidx_v,
                    didx_v, dst_v, wbuf, rows, num_sh, sem):
    cid = lax.axis_index("c")
    sid = lax.axis_index("s")
    wid = sid * 2 + cid
    zeros16 = jnp.zeros((16,), jnp.float32)
    JPH = 32 // 16 if H > 1 else C // 16

    pltpu.sync_copy(qd_hbm, qd_v)
    pltpu.sync_copy(ks_hbm, ks_v)
    pltpu.sync_copy(sidx_hbm.at[pl.ds(wid * 16, 16)], sidx_v)
    pltpu.sync_copy(didx_hbm.at[pl.ds(wid * 16, 16)], didx_v)
    pltpu.sync_copy(dst_hbm.at[pl.ds(wid * 16, 16)], dst_v)

    def zden(i, _):
        den_v[pl.ds(i * 16, 16)] = zeros16
        return _
    lax.fori_loop(0, N * H // 16, zden, None)

    def zrow(r_, _):
        for u in range(C // 16):
            rows[r_, pl.ds(u * 16, 16)] = zeros16
        return _
    lax.fori_loop(0, 128, zrow, None)
    pltpu.sync_copy(rows, num_sh.at[pl.ds(sid * 128, 128)])
    plsc.subcore_barrier()

    def chunk(mc, _):
        pltpu.async_copy(xr_hbm.at[sidx_v.at[mc]], rows, sem).wait()
        for g in range(8):
            iv = sidx_v[mc, pl.ds(g * 16, 16)]
            jv = didx_v[mc, pl.ds(g * 16, 16)]
            dn = dst_v[mc, pl.ds(g * 16, 16)]
            for h in range(H):
                hf = jnp.full((16,), h, jnp.int32)
                q = plsc.load_gather(qd_v, [jv * H + hf])
                k = plsc.load_gather(ks_v, [iv * H + hf])
                l = q + k
                l = jnp.where(l >= 0.0, l, 0.2 * l)
                w = jnp.exp(l)
                wbuf[h, pl.ds(g * 16, 16)] = w
                plsc.addupdate_scatter(den_v, [dn + h * N], w)
        def scale(e, _):
            ef = jnp.full((16,), e, jnp.int32)
            for h in range(H):
                hf = jnp.full((16,), h, jnp.int32)
                bc = plsc.load_gather(wbuf, [hf, ef])
                for u in range(JPH):
                    sl = pl.ds(h * (C // H) + u * 16, 16)
                    rows[e, sl] = rows[e, sl] * bc
            return _
        lax.fori_loop(0, 128, scale, None)
        pltpu.sync_copy(rows, num_sh.at[dst_v.at[mc]], add=True)
        return _
    lax.fori_loop(0, _NCH, chunk, None)

    plsc.subcore_barrier()
    pltpu.sync_copy(num_sh.at[pl.ds(sid * 128, 128)],
                    num_out.at[cid, pl.ds(sid * 128, 128)])
    pltpu.sync_copy(den_v, den_out.at[pl.ds(wid * N * H, N * H)])


def _edge_conv(C, H):
    mesh = plsc.VectorSubcoreMesh(core_axis_name="c", subcore_axis_name="s")
    body = functools.partial(_edge_conv_body, C, H)
    return pl.kernel(
        body,
        out_type=[
            jax.ShapeDtypeStruct((2, N, C), jnp.float32),
            jax.ShapeDtypeStruct((32 * N * H,), jnp.float32),
        ],
        mesh=mesh,
        compiler_params=pltpu.CompilerParams(needs_layout_passes=False),
        scratch_types=[
            pltpu.VMEM((R * N * H,), jnp.float32),
            pltpu.VMEM((R * N * H,), jnp.float32),
            pltpu.VMEM((N * H,), jnp.float32),
            pltpu.VMEM((16, 128), jnp.int32),
            pltpu.VMEM((16, 128), jnp.int32),
            pltpu.VMEM((16, 128), jnp.int32),
            pltpu.VMEM((H, 128), jnp.float32),
            pltpu.VMEM((128, C), jnp.float32),
            pltpu.VMEM_SHARED((N, C), jnp.float32),
            pltpu.SemaphoreType.DMA,
        ],
    )


def _rgat_sc(xrflat, q, k, sidx2, didx2, dst2, C, H):
    xr4 = xrflat.reshape(R * N, H, C // H)
    qd = jnp.einsum('nhc,hc->nh', xr4, q.reshape(H, C // H)).reshape(R * N * H)
    ks = jnp.einsum('nhc,hc->nh', xr4, k.reshape(H, C // H)).reshape(R * N * H)
    Cp = C
    if H == 1 and C < 128:
        Cp = 128
        xrflat = jnp.pad(xrflat, ((0, 0), (0, Cp - C)))
    num_parts, den_parts = _edge_conv(Cp, H)(
        sidx2, didx2, dst2, qd, ks, xrflat)
    num = num_parts[0] + num_parts[1]
    den = den_parts.reshape(32, N, H).sum(0)
    out = num.reshape(N, H, Cp // H) / (den[:, :, None] + 1e-16)
    return out.reshape(N, Cp)[:, :C]


_NB = 256
_NG = N // _NB


def _enc1_body(x_ref, W1_ref, aq_ref, ak_ref, xr_ref, qd_ref, ks_ref):
    xb = x_ref[...]
    for r in range(R):
        xr = jnp.dot(xb, W1_ref[r], preferred_element_type=jnp.float32)
        xr_ref[r] = xr
        qd_ref[r] = jnp.dot(xr, aq_ref[...], preferred_element_type=jnp.float32)
        ks_ref[r] = jnp.dot(xr, ak_ref[...], preferred_element_type=jnp.float32)


def _enc1(x, W1, Aq, Ak):
    return pl.pallas_call(
        _enc1_body,
        grid=(_NG,),
        in_specs=[
            pl.BlockSpec((_NB, D_IN), lambda i: (i, 0)),
            pl.BlockSpec((R, D_IN, HID), lambda i: (0, 0, 0)),
            pl.BlockSpec((HID, HEADS), lambda i: (0, 0)),
            pl.BlockSpec((HID, HEADS), lambda i: (0, 0)),
        ],
        out_specs=[
            pl.BlockSpec((R, _NB, HID), lambda i: (0, i, 0)),
            pl.BlockSpec((R, _NB, HEADS), lambda i: (0, i, 0)),
            pl.BlockSpec((R, _NB, HEADS), lambda i: (0, i, 0)),
        ],
        out_shape=[
            jax.ShapeDtypeStruct((R, N, HID), jnp.float32),
            jax.ShapeDtypeStruct((R, N, HEADS), jnp.float32),
            jax.ShapeDtypeStruct((R, N, HEADS), jnp.float32),
        ],
    )(x, W1, Aq, Ak)


def _bnstat_body(nump_ref, denp_ref, exp_ref, out1_ref, psum_ref, psq_ref):
    i = pl.program_id(0)
    num = nump_ref[0] + nump_ref[1]
    den = jnp.sum(denp_ref[...], axis=0)
    den128 = lax.dot_general(den, exp_ref[...], (((0,), (0,)), ((), ())),
                             preferred_element_type=jnp.float32) + 1e-16
    out1 = num / den128
    out1_ref[...] = out1

    @pl.when(i == 0)
    def _():
        psum_ref[...] = jnp.zeros_like(psum_ref)
        psq_ref[...] = jnp.zeros_like(psq_ref)
    psum_ref[...] += jnp.sum(out1, axis=0, keepdims=True)
    psq_ref[...] += jnp.sum(out1 * out1, axis=0, keepdims=True)


def _bnstat(num_parts, den_parts, expand):
    return pl.pallas_call(
        _bnstat_body,
        grid=(_NG,),
        in_specs=[
            pl.BlockSpec((2, _NB, HID), lambda i: (0, i, 0)),
            pl.BlockSpec((32, HEADS, _NB), lambda i: (0, 0, i)),
            pl.BlockSpec((HEADS, HID), lambda i: (0, 0)),
        ],
        out_specs=[
            pl.BlockSpec((_NB, HID), lambda i: (i, 0)),
            pl.BlockSpec((1, HID), lambda i: (0, 0)),
            pl.BlockSpec((1, HID), lambda i: (0, 0)),
        ],
        out_shape=[
            jax.ShapeDtypeStruct((N, HID), jnp.float32),
            jax.ShapeDtypeStruct((1, HID), jnp.float32),
            jax.ShapeDtypeStruct((1, HID), jnp.float32),
        ],
    )(num_parts, den_parts, expand)


def _enc2_body(out1_ref, psum_ref, psq_ref, g_ref, b_ref, W2_ref, aq_ref,
               ak_ref, rw_ref, h_ref, xr2_ref, qd2_ref, ks2_ref, hres_ref):
    mu = psum_ref[...] / N
    var = psq_ref[...] / N - mu * mu
    hn = (out1_ref[...] - mu) / jnp.sqrt(var + 1e-5) * g_ref[...] + b_ref[...]
    h = jnp.where(hn >= 0.0, hn, 0.2 * hn)
    h_ref[...] = h
    for r in range(R):
        xr = jnp.dot(h, W2_ref[r], preferred_element_type=jnp.float32)
        xr2_ref[r] = xr
        qd2_ref[r] = jnp.dot(xr, aq_ref[...], preferred_element_type=jnp.float32)
        ks2_ref[r] = jnp.dot(xr, ak_ref[...], preferred_element_type=jnp.float32)
    hres_ref[...] = jnp.dot(h, rw_ref[...], preferred_element_type=jnp.float32)


def _enc2(out1, psum, psq, g1, b1, W2p, Aq2, Ak2, res_W):
    return pl.pallas_call(
        _enc2_body,
        grid=(_NG,),
        in_specs=[
            pl.BlockSpec((_NB, HID), lambda i: (i, 0)),
            pl.BlockSpec((1, HID), lambda i: (0, 0)),
            pl.BlockSpec((1, HID), lambda i: (0, 0)),
            pl.BlockSpec((1, HID), lambda i: (0, 0)),
            pl.BlockSpec((1, HID), lambda i: (0, 0)),
            pl.BlockSpec((R, HID, 128), lambda i: (0, 0, 0)),
            pl.BlockSpec((128, 8), lambda i: (0, 0)),
            pl.BlockSpec((128, 8), lambda i: (0, 0)),
            pl.BlockSpec((HID, LAT), lambda i: (0, 0)),
        ],
        out_specs=[
            pl.BlockSpec((_NB, HID), lambda i: (i, 0)),
            pl.BlockSpec((R, _NB, 128), lambda i: (0, i, 0)),
            pl.BlockSpec((R, _NB, 8), lambda i: (0, i, 0)),
            pl.BlockSpec((R, _NB, 8), lambda i: (0, i, 0)),
            pl.BlockSpec((_NB, LAT), lambda i: (i, 0)),
        ],
        out_shape=[
            jax.ShapeDtypeStruct((N, HID), jnp.float32),
            jax.ShapeDtypeStruct((R, N, 128), jnp.float32),
            jax.ShapeDtypeStruct((R, N, 8), jnp.float32),
            jax.ShapeDtypeStruct((R, N, 8), jnp.float32),
            jax.ShapeDtypeStruct((N, LAT), jnp.float32),
        ],
    )(out1, psum, psq, g1.reshape(1, HID), b1.reshape(1, HID), W2p, Aq2,
      Ak2, res_W)


def _dec_body(nump_ref, denp_ref, hres_ref, rb_ref, dW1_ref, db1_ref,
              lng_ref, lnb_ref, dW2_ref, db2_ref, z_ref, xhat_ref, gmax_ref):
    i = pl.program_id(0)
    num = (nump_ref[0] + nump_ref[1])[:, :LAT]
    den = jnp.sum(denp_ref[...], axis=0)[:, None] + 1e-16
    zc = jnp.clip(num / den, -10.0, 10.0)
    z = zc + hres_ref[...] + rb_ref[...]
    z_ref[...] = z
    t = jnp.dot(z, dW1_ref[...], preferred_element_type=jnp.float32) + db1_ref[...]
    tm = jnp.mean(t, axis=1, keepdims=True)
    tv = jnp.mean(t * t, axis=1, keepdims=True) - tm * tm
    t = (t - tm) / jnp.sqrt(tv + 1e-5) * lng_ref[...] + lnb_ref[...]
    t = jnp.maximum(t, 0.0)
    xhat_ref[...] = jnp.dot(t, dW2_ref[...],
                            preferred_element_type=jnp.float32) + db2_ref[...]
    bmax = jnp.max(z, axis=0, keepdims=True)

    @pl.when(i == 0)
    def _():
        gmax_ref[...] = bmax

    @pl.when(i > 0)
    def _():
        gmax_ref[...] = jnp.maximum(gmax_ref[...], bmax)


def _dec(num_parts, den_parts, hres, res_b, dec_W1, dec_b1, ln_g, ln_b,
         dec_W2, dec_b2):
    return pl.pallas_call(
        _dec_body,
        grid=(_NG,),
        in_specs=[
            pl.BlockSpec((2, _NB, 128), lambda i: (0, i, 0)),
            pl.BlockSpec((32, _NB), lambda i: (0, i)),
            pl.BlockSpec((_NB, LAT), lambda i: (i, 0)),
            pl.BlockSpec((1, LAT), lambda i: (0, 0)),
            pl.BlockSpec((LAT, HID), lambda i: (0, 0)),
            pl.BlockSpec((1, HID), lambda i: (0, 0)),
            pl.BlockSpec((1, HID), lambda i: (0, 0)),
            pl.BlockSpec((1, HID), lambda i: (0, 0)),
            pl.BlockSpec((HID, D_IN), lambda i: (0, 0)),
            pl.BlockSpec((1, D_IN), lambda i: (0, 0)),
        ],
        out_specs=[
            pl.BlockSpec((_NB, LAT), lambda i: (i, 0)),
            pl.BlockSpec((_NB, D_IN), lambda i: (i, 0)),
            pl.BlockSpec((1, LAT), lambda i: (0, 0)),
        ],
        out_shape=[
            jax.ShapeDtypeStruct((N, LAT), jnp.float32),
            jax.ShapeDtypeStruct((N, D_IN), jnp.float32),
            jax.ShapeDtypeStruct((1, LAT), jnp.float32),
        ],
    )(num_parts, den_parts, hres, res_b.reshape(1, LAT), dec_W1,
      dec_b1.reshape(1, HID), ln_g.reshape(1, HID), ln_b.reshape(1, HID),
      dec_W2, dec_b2.reshape(1, D_IN))


def kernel(x, edge_index, edge_type, W1, q1, k1, g1, b1, W2, q2, k2,
           res_W, res_b, dec_W1, dec_b1, ln_g, ln_b, dec_W2, dec_b2,
           rel_diag):
    src = edge_index[0]
    dst = edge_index[1]
    f, sidx2, didx2 = _edge_addrs(src, dst, edge_type)
    dst2 = dst.reshape(512, 128)
    adj_true_rel = _adj_true(f)

    eye4 = jnp.eye(HEADS, dtype=jnp.float32)
    Aq1 = (q1[:, :, None] * eye4[:, None, :]).reshape(HID, HEADS)
    Ak1 = (k1[:, :, None] * eye4[:, None, :]).reshape(HID, HEADS)
    xr1, qd1, ks1 = _enc1(x, W1, Aq1, Ak1)
    num1p, den1p = _edge_conv(HID, HEADS)(
        sidx2, didx2, dst2, qd1.reshape(R * N * HEADS),
        ks1.reshape(R * N * HEADS), xr1.reshape(R * N, HID))

    expand = jnp.repeat(eye4, HID // HEADS, axis=1)
    out1, psum, psq = _bnstat(num1p, den1p.reshape(32, HEADS, N), expand)

    W2p = jnp.pad(W2, ((0, 0), (0, 0), (0, 128 - LAT)))
    Aq2 = jnp.zeros((128, 8), jnp.float32).at[:LAT, 0].set(q2[0])
    Ak2 = jnp.zeros((128, 8), jnp.float32).at[:LAT, 0].set(k2[0])
    h, xr2p, qd2, ks2, hres = _enc2(out1, psum, psq, g1, b1, W2p, Aq2,
                                    Ak2, res_W)
    num2p, den2p = _edge_conv(128, 1)(
        sidx2, didx2, dst2, qd2[:, :, 0].reshape(R * N),
        ks2[:, :, 0].reshape(R * N), xr2p.reshape(R * N, 128))

    z, x_hat, graph_embedding = _dec(
        num2p, den2p.reshape(32, N), hres, res_b, dec_W1, dec_b1, ln_g,
        ln_b, dec_W2, dec_b2)

    mask = jnp.ones((1, N), dtype=bool)
    adj_preds = _adj_preds(z, rel_diag)
    return (z, x_hat, adj_preds, adj_true_rel, mask, graph_embedding)


# speedup vs baseline: 1.3975x; 1.1041x over previous
import functools

import jax
import jax.numpy as jnp
from jax import lax
from jax.experimental import pallas as pl
from jax.experimental.pallas import tpu as pltpu
from jax.experimental.pallas import tpu_sc as plsc

N = 2048
E = 65536
D_IN = 256
R = 3
HID = 128
LAT = 64
HEADS = 4

ROW_BLK = 256


def _adj_pred_body(z_blk_ref, zt_ref, rd_ref, out_ref):
    z_blk = z_blk_ref[...]
    zt = zt_ref[...]
    for r in range(R):
        zr = z_blk * rd_ref[r][None, :]
        score = jnp.dot(zr, zt, preferred_element_type=jnp.float32)
        score = jnp.clip(score, -10.0, 10.0)
        p = jax.nn.sigmoid(score)
        out_ref[r, 0] = jnp.clip(p, 1e-6, 1.0 - 1e-6)


def _adj_preds(z, rel_diag):
    zt = z.T
    grid = (N // ROW_BLK,)
    return pl.pallas_call(
        _adj_pred_body,
        grid=grid,
        in_specs=[
            pl.BlockSpec((ROW_BLK, LAT), lambda i: (i, 0)),
            pl.BlockSpec((LAT, N), lambda i: (0, 0)),
            pl.BlockSpec((R, LAT), lambda i: (0, 0)),
        ],
        out_specs=pl.BlockSpec((R, 1, ROW_BLK, N), lambda i: (0, 0, i, 0)),
        out_shape=jax.ShapeDtypeStruct((R, 1, N, N), jnp.float32),
    )(z, zt, rel_diag)


def _edge_addr_body(src_ref, dst_ref, et_ref, f_ref, sidx_ref, didx_ref):
    base = et_ref[...] * N
    sidx_ref[...] = base + src_ref[...]
    didx_ref[...] = base + dst_ref[...]
    f_ref[...] = sidx_ref[...] * N + dst_ref[...]


def _edge_addrs(src, dst, et):
    s2 = src.reshape(64, 1024)
    d2 = dst.reshape(64, 1024)
    t2 = et.reshape(64, 1024)
    f, sidx, didx = pl.pallas_call(
        _edge_addr_body,
        out_shape=[jax.ShapeDtypeStruct((64, 1024), jnp.int32)] * 3,
    )(s2, d2, t2)
    return f.reshape(E), sidx.reshape(512, 128), didx.reshape(512, 128)


_ADJ_ROWS = 48
_ADJ_PAGE = _ADJ_ROWS * N
_ADJ_ROUNDS = (R * N * N) // (32 * _ADJ_PAGE)
_EBLK = 8192


def _adj_true_body(f_hbm, out_hbm, page, fbufs, sem0, sem1):
    wid = lax.axis_index("s") * 2 + lax.axis_index("c")
    ones = jnp.ones((16,), jnp.float32)
    zeros16 = jnp.zeros((16,), jnp.float32)

    def round_body(k, _):
        base = (k * 32 + wid) * _ADJ_PAGE

        def zero_chunk(i, _):
            for u in range(8):
                page[(i * 128 + u * 16) // 2048, pl.ds((i * 128 + u * 16) % 2048, 16)] = zeros16
            return _
        lax.fori_loop(0, _ADJ_PAGE // 128, zero_chunk, None)
        cp0 = pltpu.async_copy(f_hbm.at[pl.ds(0, _EBLK)], fbufs.at[0], sem0)
        for b in range(E // _EBLK):
            cur = b % 2
            if b % 2 == 0:
                cp0.wait()
            else:
                cp1.wait()
            if b + 1 < E // _EBLK:
                nxt = (b + 1) % 2
                if nxt == 0:
                    cp0 = pltpu.async_copy(
                        f_hbm.at[pl.ds((b + 1) * _EBLK, _EBLK)],
                        fbufs.at[0], sem0)
                else:
                    cp1 = pltpu.async_copy(
                        f_hbm.at[pl.ds((b + 1) * _EBLK, _EBLK)],
                        fbufs.at[1], sem1)

            def chunk(i, _):
                for u in range(8):
                    fv = fbufs[cur, pl.ds(i * 128 + u * 16, 16)]
                    local = fv - base
                    m = plsc.bitcast(local, jnp.uint32) < jnp.uint32(_ADJ_PAGE)
                    plsc.addupdate_scatter(
                        page, [local >> 11, local & 2047], ones, mask=m)
                return _
            lax.fori_loop(0, _EBLK // 128, chunk, None)
        rbase = (k * 32 + wid) * _ADJ_ROWS
        for sb in range(_ADJ_ROWS // 16):
            rb = rbase + sb * 16
            rel = rb // N
            srow = rb - rel * N
            pltpu.sync_copy(page.at[pl.ds(sb * 16, 16)],
                            out_hbm.at[rel, 0, pl.ds(srow, 16), :])
        return _
    lax.fori_loop(0, _ADJ_ROUNDS, round_body, None)


def _adj_true(f):
    mesh = plsc.VectorSubcoreMesh(core_axis_name="c", subcore_axis_name="s")
    run = pl.kernel(
        _adj_true_body,
        out_type=jax.ShapeDtypeStruct((R, 1, N, N), jnp.float32),
        mesh=mesh,
        compiler_params=pltpu.CompilerParams(needs_layout_passes=False),
        scratch_types=[
            pltpu.VMEM((_ADJ_ROWS, N), jnp.float32),
            pltpu.VMEM((2, _EBLK), jnp.int32),
            pltpu.SemaphoreType.DMA,
            pltpu.SemaphoreType.DMA,
        ],
    )
    return run(f)


_EPT = E // 32
_NCH = _EPT // 128


def _edge_conv_body(C, H, sidx_hbm, didx_hbm, dst_hbm, qd_hbm, ks_hbm,
                    xr_hbm, num_out, den_out, qd_v, ks_v, den_v, sidx_v,
                    didx_v, dst_v, wbuf, rows, num_sh, sem, sem2):
    cid = lax.axis_index("c")
    sid = lax.axis_index("s")
    wid = sid * 2 + cid
    zeros16 = jnp.zeros((16,), jnp.float32)
    JPH = 32 // 16 if H > 1 else C // 16

    pltpu.sync_copy(qd_hbm, qd_v)
    pltpu.sync_copy(ks_hbm, ks_v)
    pltpu.sync_copy(sidx_hbm.at[pl.ds(wid * 16, 16)], sidx_v)
    pltpu.sync_copy(didx_hbm.at[pl.ds(wid * 16, 16)], didx_v)
    pltpu.sync_copy(dst_hbm.at[pl.ds(wid * 16, 16)], dst_v)

    def zden(i, _):
        den_v[pl.ds(i * 16, 16)] = zeros16
        return _
    lax.fori_loop(0, N * H // 16, zden, None)

    def zrow(r_, _):
        for u in range(C // 16):
            rows[0, r_, pl.ds(u * 16, 16)] = zeros16
        return _
    lax.fori_loop(0, 128, zrow, None)
    pltpu.sync_copy(rows.at[0], num_sh.at[pl.ds(sid * 128, 128)])
    plsc.subcore_barrier()

    pltpu.async_copy(xr_hbm.at[sidx_v.at[0]], rows.at[0], sem)
    pltpu.async_copy(xr_hbm.at[sidx_v.at[1]], rows.at[1], sem2)

    def chunk2(j, _):
        for p in range(2):
            mc = j * 2 + p
            sm = sem if p == 0 else sem2
            buf = rows.at[p]
            for g in range(8):
                iv = sidx_v[mc, pl.ds(g * 16, 16)]
                jv = didx_v[mc, pl.ds(g * 16, 16)]
                dn = dst_v[mc, pl.ds(g * 16, 16)]
                for h in range(H):
                    hf = jnp.full((16,), h, jnp.int32)
                    q = plsc.load_gather(qd_v, [jv * H + hf])
                    k = plsc.load_gather(ks_v, [iv * H + hf])
                    l = q + k
                    l = jnp.where(l >= 0.0, l, 0.2 * l)
                    w = jnp.exp(l)
                    wbuf[h, pl.ds(g * 16, 16)] = w
                    plsc.addupdate_scatter(den_v, [dn + h * N], w)
            pltpu.make_async_copy(xr_hbm.at[sidx_v.at[mc]], buf, sm).wait()

            def scale(e2, _):
                for v in range(2):
                    e = e2 * 2 + v
                    ef = jnp.full((16,), e, jnp.int32)
                    for h in range(H):
                        hf = jnp.full((16,), h, jnp.int32)
                        bc = plsc.load_gather(wbuf, [hf, ef])
                        for u in range(JPH):
                            sl = pl.ds(h * (C // H) + u * 16, 16)
                            rows[p, e, sl] = rows[p, e, sl] * bc
                return _
            lax.fori_loop(0, 64, scale, None)
            pltpu.sync_copy(buf, num_sh.at[dst_v.at[mc]], add=True)

            @pl.when(mc + 2 < _NCH)
            def _():
                pltpu.async_copy(xr_hbm.at[sidx_v.at[mc + 2]], buf, sm)
        return _
    lax.fori_loop(0, _NCH // 2, chunk2, None)

    plsc.subcore_barrier()
    pltpu.sync_copy(num_sh.at[pl.ds(sid * 128, 128)],
                    num_out.at[cid, pl.ds(sid * 128, 128)])
    pltpu.sync_copy(den_v, den_out.at[pl.ds(wid * N * H, N * H)])


def _edge_conv(C, H):
    mesh = plsc.VectorSubcoreMesh(core_axis_name="c", subcore_axis_name="s")
    body = functools.partial(_edge_conv_body, C, H)
    return pl.kernel(
        body,
        out_type=[
            jax.ShapeDtypeStruct((2, N, C), jnp.float32),
            jax.ShapeDtypeStruct((32 * N * H,), jnp.float32),
        ],
        mesh=mesh,
        compiler_params=pltpu.CompilerParams(needs_layout_passes=False),
        scratch_types=[
            pltpu.VMEM((R * N * H,), jnp.float32),
            pltpu.VMEM((R * N * H,), jnp.float32),
            pltpu.VMEM((N * H,), jnp.float32),
            pltpu.VMEM((16, 128), jnp.int32),
            pltpu.VMEM((16, 128), jnp.int32),
            pltpu.VMEM((16, 128), jnp.int32),
            pltpu.VMEM((H, 128), jnp.float32),
            pltpu.VMEM((2, 128, C), jnp.float32),
            pltpu.VMEM_SHARED((N, C), jnp.float32),
            pltpu.SemaphoreType.DMA,
            pltpu.SemaphoreType.DMA,
        ],
    )


def _rgat_sc(xrflat, q, k, sidx2, didx2, dst2, C, H):
    xr4 = xrflat.reshape(R * N, H, C // H)
    qd = jnp.einsum('nhc,hc->nh', xr4, q.reshape(H, C // H)).reshape(R * N * H)
    ks = jnp.einsum('nhc,hc->nh', xr4, k.reshape(H, C // H)).reshape(R * N * H)
    Cp = C
    if H == 1 and C < 128:
        Cp = 128
        xrflat = jnp.pad(xrflat, ((0, 0), (0, Cp - C)))
    num_parts, den_parts = _edge_conv(Cp, H)(
        sidx2, didx2, dst2, qd, ks, xrflat)
    num = num_parts[0] + num_parts[1]
    den = den_parts.reshape(32, N, H).sum(0)
    out = num.reshape(N, H, Cp // H) / (den[:, :, None] + 1e-16)
    return out.reshape(N, Cp)[:, :C]


_NB = 256
_NG = N // _NB


def _enc1_body(x_ref, W1_ref, aq_ref, ak_ref, xr_ref, qd_ref, ks_ref):
    xb = x_ref[...]
    for r in range(R):
        xr = jnp.dot(xb, W1_ref[r], preferred_element_type=jnp.float32)
        xr_ref[r] = xr
        qd_ref[r] = jnp.dot(xr, aq_ref[...], preferred_element_type=jnp.float32)
        ks_ref[r] = jnp.dot(xr, ak_ref[...], preferred_element_type=jnp.float32)


def _enc1(x, W1, Aq, Ak):
    return pl.pallas_call(
        _enc1_body,
        grid=(_NG,),
        in_specs=[
            pl.BlockSpec((_NB, D_IN), lambda i: (i, 0)),
            pl.BlockSpec((R, D_IN, HID), lambda i: (0, 0, 0)),
            pl.BlockSpec((HID, HEADS), lambda i: (0, 0)),
            pl.BlockSpec((HID, HEADS), lambda i: (0, 0)),
        ],
        out_specs=[
            pl.BlockSpec((R, _NB, HID), lambda i: (0, i, 0)),
            pl.BlockSpec((R, _NB, HEADS), lambda i: (0, i, 0)),
            pl.BlockSpec((R, _NB, HEADS), lambda i: (0, i, 0)),
        ],
        out_shape=[
            jax.ShapeDtypeStruct((R, N, HID), jnp.float32),
            jax.ShapeDtypeStruct((R, N, HEADS), jnp.float32),
            jax.ShapeDtypeStruct((R, N, HEADS), jnp.float32),
        ],
    )(x, W1, Aq, Ak)


def _bnstat_body(nump_ref, denp_ref, exp_ref, out1_ref, psum_ref, psq_ref):
    i = pl.program_id(0)
    num = nump_ref[0] + nump_ref[1]
    den = jnp.sum(denp_ref[...], axis=0)
    den128 = lax.dot_general(den, exp_ref[...], (((0,), (0,)), ((), ())),
                             preferred_element_type=jnp.float32) + 1e-16
    out1 = num / den128
    out1_ref[...] = out1

    @pl.when(i == 0)
    def _():
        psum_ref[...] = jnp.zeros_like(psum_ref)
        psq_ref[...] = jnp.zeros_like(psq_ref)
    psum_ref[...] += jnp.sum(out1, axis=0, keepdims=True)
    psq_ref[...] += jnp.sum(out1 * out1, axis=0, keepdims=True)


def _bnstat(num_parts, den_parts, expand):
    return pl.pallas_call(
        _bnstat_body,
        grid=(_NG,),
        in_specs=[
            pl.BlockSpec((2, _NB, HID), lambda i: (0, i, 0)),
            pl.BlockSpec((32, HEADS, _NB), lambda i: (0, 0, i)),
            pl.BlockSpec((HEADS, HID), lambda i: (0, 0)),
        ],
        out_specs=[
            pl.BlockSpec((_NB, HID), lambda i: (i, 0)),
            pl.BlockSpec((1, HID), lambda i: (0, 0)),
            pl.BlockSpec((1, HID), lambda i: (0, 0)),
        ],
        out_shape=[
            jax.ShapeDtypeStruct((N, HID), jnp.float32),
            jax.ShapeDtypeStruct((1, HID), jnp.float32),
            jax.ShapeDtypeStruct((1, HID), jnp.float32),
        ],
    )(num_parts, den_parts, expand)


def _enc2_body(out1_ref, psum_ref, psq_ref, g_ref, b_ref, W2_ref, aq_ref,
               ak_ref, rw_ref, h_ref, xr2_ref, qd2_ref, ks2_ref, hres_ref):
    mu = psum_ref[...] / N
    var = psq_ref[...] / N - mu * mu
    hn = (out1_ref[...] - mu) / jnp.sqrt(var + 1e-5) * g_ref[...] + b_ref[...]
    h = jnp.where(hn >= 0.0, hn, 0.2 * hn)
    h_ref[...] = h
    for r in range(R):
        xr = jnp.dot(h, W2_ref[r], preferred_element_type=jnp.float32)
        xr2_ref[r] = xr
        qd2_ref[r] = jnp.dot(xr, aq_ref[...], preferred_element_type=jnp.float32)
        ks2_ref[r] = jnp.dot(xr, ak_ref[...], preferred_element_type=jnp.float32)
    hres_ref[...] = jnp.dot(h, rw_ref[...], preferred_element_type=jnp.float32)


def _enc2(out1, psum, psq, g1, b1, W2p, Aq2, Ak2, res_W):
    return pl.pallas_call(
        _enc2_body,
        grid=(_NG,),
        in_specs=[
            pl.BlockSpec((_NB, HID), lambda i: (i, 0)),
            pl.BlockSpec((1, HID), lambda i: (0, 0)),
            pl.BlockSpec((1, HID), lambda i: (0, 0)),
            pl.BlockSpec((1, HID), lambda i: (0, 0)),
            pl.BlockSpec((1, HID), lambda i: (0, 0)),
            pl.BlockSpec((R, HID, 128), lambda i: (0, 0, 0)),
            pl.BlockSpec((128, 8), lambda i: (0, 0)),
            pl.BlockSpec((128, 8), lambda i: (0, 0)),
            pl.BlockSpec((HID, LAT), lambda i: (0, 0)),
        ],
        out_specs=[
            pl.BlockSpec((_NB, HID), lambda i: (i, 0)),
            pl.BlockSpec((R, _NB, 128), lambda i: (0, i, 0)),
            pl.BlockSpec((R, _NB, 8), lambda i: (0, i, 0)),
            pl.BlockSpec((R, _NB, 8), lambda i: (0, i, 0)),
            pl.BlockSpec((_NB, LAT), lambda i: (i, 0)),
        ],
        out_shape=[
            jax.ShapeDtypeStruct((N, HID), jnp.float32),
            jax.ShapeDtypeStruct((R, N, 128), jnp.float32),
            jax.ShapeDtypeStruct((R, N, 8), jnp.float32),
            jax.ShapeDtypeStruct((R, N, 8), jnp.float32),
            jax.ShapeDtypeStruct((N, LAT), jnp.float32),
        ],
    )(out1, psum, psq, g1.reshape(1, HID), b1.reshape(1, HID), W2p, Aq2,
      Ak2, res_W)


def _dec_body(nump_ref, denp_ref, hres_ref, rb_ref, dW1_ref, db1_ref,
              lng_ref, lnb_ref, dW2_ref, db2_ref, z_ref, xhat_ref, gmax_ref):
    i = pl.program_id(0)
    num = (nump_ref[0] + nump_ref[1])[:, :LAT]
    den = jnp.sum(denp_ref[...], axis=0)[:, None] + 1e-16
    zc = jnp.clip(num / den, -10.0, 10.0)
    z = zc + hres_ref[...] + rb_ref[...]
    z_ref[...] = z
    t = jnp.dot(z, dW1_ref[...], preferred_element_type=jnp.float32) + db1_ref[...]
    tm = jnp.mean(t, axis=1, keepdims=True)
    tv = jnp.mean(t * t, axis=1, keepdims=True) - tm * tm
    t = (t - tm) / jnp.sqrt(tv + 1e-5) * lng_ref[...] + lnb_ref[...]
    t = jnp.maximum(t, 0.0)
    xhat_ref[...] = jnp.dot(t, dW2_ref[...],
                            preferred_element_type=jnp.float32) + db2_ref[...]
    bmax = jnp.max(z, axis=0, keepdims=True)

    @pl.when(i == 0)
    def _():
        gmax_ref[...] = bmax

    @pl.when(i > 0)
    def _():
        gmax_ref[...] = jnp.maximum(gmax_ref[...], bmax)


def _dec(num_parts, den_parts, hres, res_b, dec_W1, dec_b1, ln_g, ln_b,
         dec_W2, dec_b2):
    return pl.pallas_call(
        _dec_body,
        grid=(_NG,),
        in_specs=[
            pl.BlockSpec((2, _NB, 128), lambda i: (0, i, 0)),
            pl.BlockSpec((32, _NB), lambda i: (0, i)),
            pl.BlockSpec((_NB, LAT), lambda i: (i, 0)),
            pl.BlockSpec((1, LAT), lambda i: (0, 0)),
            pl.BlockSpec((LAT, HID), lambda i: (0, 0)),
            pl.BlockSpec((1, HID), lambda i: (0, 0)),
            pl.BlockSpec((1, HID), lambda i: (0, 0)),
            pl.BlockSpec((1, HID), lambda i: (0, 0)),
            pl.BlockSpec((HID, D_IN), lambda i: (0, 0)),
            pl.BlockSpec((1, D_IN), lambda i: (0, 0)),
        ],
        out_specs=[
            pl.BlockSpec((_NB, LAT), lambda i: (i, 0)),
            pl.BlockSpec((_NB, D_IN), lambda i: (i, 0)),
            pl.BlockSpec((1, LAT), lambda i: (0, 0)),
        ],
        out_shape=[
            jax.ShapeDtypeStruct((N, LAT), jnp.float32),
            jax.ShapeDtypeStruct((N, D_IN), jnp.float32),
            jax.ShapeDtypeStruct((1, LAT), jnp.float32),
        ],
    )(num_parts, den_parts, hres, res_b.reshape(1, LAT), dec_W1,
      dec_b1.reshape(1, HID), ln_g.reshape(1, HID), ln_b.reshape(1, HID),
      dec_W2, dec_b2.reshape(1, D_IN))


def kernel(x, edge_index, edge_type, W1, q1, k1, g1, b1, W2, q2, k2,
           res_W, res_b, dec_W1, dec_b1, ln_g, ln_b, dec_W2, dec_b2,
           rel_diag):
    src = edge_index[0]
    dst = edge_index[1]
    f, sidx2, didx2 = _edge_addrs(src, dst, edge_type)
    dst2 = dst.reshape(512, 128)
    adj_true_rel = _adj_true(f)

    eye4 = jnp.eye(HEADS, dtype=jnp.float32)
    Aq1 = (q1[:, :, None] * eye4[:, None, :]).reshape(HID, HEADS)
    Ak1 = (k1[:, :, None] * eye4[:, None, :]).reshape(HID, HEADS)
    xr1, qd1, ks1 = _enc1(x, W1, Aq1, Ak1)
    num1p, den1p = _edge_conv(HID, HEADS)(
        sidx2, didx2, dst2, qd1.reshape(R * N * HEADS),
        ks1.reshape(R * N * HEADS), xr1.reshape(R * N, HID))

    expand = jnp.repeat(eye4, HID // HEADS, axis=1)
    out1, psum, psq = _bnstat(num1p, den1p.reshape(32, HEADS, N), expand)

    W2p = jnp.pad(W2, ((0, 0), (0, 0), (0, 128 - LAT)))
    Aq2 = jnp.zeros((128, 8), jnp.float32).at[:LAT, 0].set(q2[0])
    Ak2 = jnp.zeros((128, 8), jnp.float32).at[:LAT, 0].set(k2[0])
    h, xr2p, qd2, ks2, hres = _enc2(out1, psum, psq, g1, b1, W2p, Aq2,
                                    Ak2, res_W)
    num2p, den2p = _edge_conv(128, 1)(
        sidx2, didx2, dst2, qd2[:, :, 0].reshape(R * N),
        ks2[:, :, 0].reshape(R * N), xr2p.reshape(R * N, 128))

    z, x_hat, graph_embedding = _dec(
        num2p, den2p.reshape(32, N), hres, res_b, dec_W1, dec_b1, ln_g,
        ln_b, dec_W2, dec_b2)

    mask = jnp.ones((1, N), dtype=bool)
    adj_preds = _adj_preds(z, rel_diag)
    return (z, x_hat, adj_preds, adj_true_rel, mask, graph_embedding)
